# baseline (speedup 1.0000x reference)
import functools

import jax
import jax.numpy as jnp
from jax import lax
from jax.experimental import pallas as pl
from jax.experimental.pallas import tpu as pltpu
from jax.experimental.pallas import tpu_sc as plsc

N = 10000
E = 640000
IN_NF = 128
HID = 32
A_NF = 16
XP = 16
BE = 6400
BN = 2000

NC = 2
NS = 16
NW = NC * NS
EPW = E // NW
GC = 800
GK = 80
SB = 80
SJ = 10
SCC = SB * SJ


def _gather_sc(ht, xt, dst, src):
    mesh = plsc.VectorSubcoreMesh(core_axis_name="c", subcore_axis_name="s")

    @functools.partial(
        pl.kernel, mesh=mesh,
        out_type=[jax.ShapeDtypeStruct((E, HID), jnp.float32),
                  jax.ShapeDtypeStruct((E, HID), jnp.float32),
                  jax.ShapeDtypeStruct((E, XP), jnp.float32),
                  jax.ShapeDtypeStruct((E, XP), jnp.float32)],
        scratch_types=[pltpu.VMEM((GC,), jnp.int32),
                       pltpu.VMEM((GC,), jnp.int32),
                       pltpu.VMEM((GC, HID), jnp.float32),
                       pltpu.VMEM((GC, HID), jnp.float32),
                       pltpu.VMEM((GC, XP), jnp.float32),
                       pltpu.VMEM((GC, XP), jnp.float32),
                       pltpu.SemaphoreType.DMA],
        compiler_params=pltpu.CompilerParams(use_tc_tiling_on_sc=False),
    )
    def k(ht_hbm, xt_hbm, dst_hbm, src_hbm,
          hi_hbm, hj_hbm, xi_hbm, xj_hbm,
          idxd_v, idxs_v, hd_v, hs_v, xd_v, xs_v, sem):
        wid = lax.axis_index("s") * NC + lax.axis_index("c")
        base0 = wid * EPW

        def body(t, carry):
            base = base0 + t * GC
            pltpu.sync_copy(dst_hbm.at[pl.ds(base, GC)], idxd_v)
            pltpu.sync_copy(src_hbm.at[pl.ds(base, GC)], idxs_v)
            cps = []
            for j in range(GC // GK):
                s = pl.ds(j * GK, GK)
                cps.append(pltpu.async_copy(ht_hbm.at[idxd_v.at[s]], hd_v.at[s], sem))
                cps.append(pltpu.async_copy(ht_hbm.at[idxs_v.at[s]], hs_v.at[s], sem))
                cps.append(pltpu.async_copy(xt_hbm.at[idxd_v.at[s]], xd_v.at[s], sem))
                cps.append(pltpu.async_copy(xt_hbm.at[idxs_v.at[s]], xs_v.at[s], sem))
            for cp in cps:
                cp.wait()
            pltpu.sync_copy(hd_v, hi_hbm.at[pl.ds(base, GC)])
            pltpu.sync_copy(hs_v, hj_hbm.at[pl.ds(base, GC)])
            pltpu.sync_copy(xd_v, xi_hbm.at[pl.ds(base, GC)])
            pltpu.sync_copy(xs_v, xj_hbm.at[pl.ds(base, GC)])
            return carry

        lax.fori_loop(0, EPW // GC, body, 0)

    return k(ht, xt, dst, src)


def _scatter_sc(mx, mh, dst3d, zro16, zro32):
    mesh = plsc.VectorSubcoreMesh(core_axis_name="c", subcore_axis_name="s")

    @functools.partial(
        pl.kernel, mesh=mesh,
        out_type=[jax.ShapeDtypeStruct((NC, N, XP), jnp.float32),
                  jax.ShapeDtypeStruct((NC, N, HID), jnp.float32)],
        scratch_types=[pltpu.VMEM((SJ, SB), jnp.int32),
                       pltpu.VMEM((SCC, XP), jnp.float32),
                       pltpu.VMEM((SCC, HID), jnp.float32),
                       pltpu.VMEM_SHARED((N, XP), jnp.float32),
                       pltpu.VMEM_SHARED((N, HID), jnp.float32)],
        compiler_params=pltpu.CompilerParams(use_tc_tiling_on_sc=False),
    )
    def k(mx_hbm, mh_hbm, dst_hbm, z16_hbm, z32_hbm, o16_hbm, o32_hbm,
          idx_v, mx_v, mh_v, sh16, sh32):
        cid = lax.axis_index("c")
        sid = lax.axis_index("s")
        wid = sid * NC + cid

        @pl.when(sid < 10)
        def _():
            pltpu.sync_copy(z16_hbm, sh16.at[pl.ds(sid * 1000, 1000)])
            pltpu.sync_copy(z32_hbm, sh32.at[pl.ds(sid * 1000, 1000)])

        plsc.subcore_barrier()

        def body(t, carry):
            blk = wid * (EPW // SCC) + t
            base = wid * EPW + t * SCC
            pltpu.sync_copy(dst_hbm.at[blk], idx_v)
            pltpu.sync_copy(mx_hbm.at[pl.ds(base, SCC)], mx_v)
            pltpu.sync_copy(mh_hbm.at[pl.ds(base, SCC)], mh_v)
            for j in range(SJ):
                s = pl.ds(j * SB, SB)
                pltpu.sync_copy(mx_v.at[s], sh16.at[idx_v.at[j]], add=True)
                pltpu.sync_copy(mh_v.at[s], sh32.at[idx_v.at[j]], add=True)
            return carry

        lax.fori_loop(0, EPW // SCC, body, 0)
        plsc.subcore_barrier()

        @pl.when(sid < 10)
        def _():
            pltpu.sync_copy(sh16.at[pl.ds(sid * 1000, 1000)],
                            o16_hbm.at[cid, pl.ds(sid * 1000, 1000)])
            pltpu.sync_copy(sh32.at[pl.ds(sid * 1000, 1000)],
                            o32_hbm.at[cid, pl.ds(sid * 1000, 1000)])

    return k(mx, mh, dst3d, zro16, zro32)


def _degree_sc(src3d, ones, zro16):
    mesh = plsc.VectorSubcoreMesh(core_axis_name="c", subcore_axis_name="s")

    @functools.partial(
        pl.kernel, mesh=mesh,
        out_type=jax.ShapeDtypeStruct((NC, N, XP), jnp.float32),
        scratch_types=[pltpu.VMEM((SJ, SB), jnp.int32),
                       pltpu.VMEM((SB, XP), jnp.float32),
                       pltpu.VMEM_SHARED((N, XP), jnp.float32)],
        compiler_params=pltpu.CompilerParams(use_tc_tiling_on_sc=False),
    )
    def k(src_hbm, ones_hbm, zro_hbm, out_hbm, idx_v, ones_v, shared):
        cid = lax.axis_index("c")
        sid = lax.axis_index("s")
        wid = sid * NC + cid
        pltpu.sync_copy(ones_hbm, ones_v)

        @pl.when(sid < 10)
        def _():
            pltpu.sync_copy(zro_hbm, shared.at[pl.ds(sid * 1000, 1000)])

        plsc.subcore_barrier()

        def body(t, carry):
            blk = wid * (EPW // SCC) + t
            pltpu.sync_copy(src_hbm.at[blk], idx_v)
            for j in range(SJ):
                pltpu.sync_copy(ones_v, shared.at[idx_v.at[j]], add=True)
            return carry

        lax.fori_loop(0, EPW // SCC, body, 0)
        plsc.subcore_barrier()

        @pl.when(sid < 10)
        def _():
            pltpu.sync_copy(shared.at[pl.ds(sid * 1000, 1000)],
                            out_hbm.at[cid, pl.ds(sid * 1000, 1000)])

    return k(src3d, ones, zro16)


def _sigmoid(z):
    return 1.0 / (1.0 + jnp.exp(-z))


def _silu(z):
    return z * _sigmoid(z)


def _ln(z, g, b, eps=1e-5):
    o = jnp.full((HID, 1), 1.0 / HID, jnp.float32)
    mu = jnp.dot(z, o, preferred_element_type=jnp.float32)
    msq = jnp.dot(z * z, o, preferred_element_type=jnp.float32)
    var = msq - mu * mu
    return (z - mu) * jax.lax.rsqrt(var + eps) * g + b


def _edge_kernel(hi_ref, hj_ref, xi_ref, xj_ref, ea_ref,
                 w1hi_ref, w1hj_ref, w1ea_ref, w1d2_ref, b1_ref, g1_ref, be1_ref,
                 w2_ref, b2_ref, g2_ref, be2_ref,
                 wx1_ref, bx1_ref, gx_ref, bex_ref, wx2_ref, bx2_ref,
                 mx_ref, mh_ref):
    hi = hi_ref[...]
    hj = hj_ref[...]
    diff = xi_ref[...] - xj_ref[...]
    o16 = jnp.full((XP, 1), 1.0, jnp.float32)
    d2 = jnp.dot(diff * diff, o16, preferred_element_type=jnp.float32)
    z = (jnp.dot(hi, w1hi_ref[...], preferred_element_type=jnp.float32)
         + jnp.dot(hj, w1hj_ref[...], preferred_element_type=jnp.float32)
         + jnp.dot(ea_ref[...], w1ea_ref[...], preferred_element_type=jnp.float32)
         + d2 * w1d2_ref[...]
         + b1_ref[...])
    z = _silu(_ln(z, g1_ref[...], be1_ref[...]))
    z = jnp.dot(z, w2_ref[...], preferred_element_type=jnp.float32) + b2_ref[...]
    mh = _silu(_ln(z, g2_ref[...], be2_ref[...]))
    t = jnp.dot(mh, wx1_ref[...], preferred_element_type=jnp.float32) + bx1_ref[...]
    t = _silu(_ln(t, gx_ref[...], bex_ref[...]))
    px = jnp.dot(t, wx2_ref[...], preferred_element_type=jnp.float32) + bx2_ref[...]
    mx_ref[...] = diff * px
    mh_ref[...] = mh


def _edge_mlp(hi, hj, xi, xj, ea, p):
    w1 = p["e1"]["W"]
    ops = dict(
        w1hi=w1[:HID], w1hj=w1[HID:2 * HID], w1ea=w1[2 * HID + 1:],
        w1d2=w1[2 * HID:2 * HID + 1], b1=p["e1"]["b"][None, :],
        g1=p["e_ln1"]["g"][None, :], be1=p["e_ln1"]["b"][None, :],
        w2=p["e2"]["W"], b2=p["e2"]["b"][None, :],
        g2=p["e_ln2"]["g"][None, :], be2=p["e_ln2"]["b"][None, :],
        wx1=p["x1"]["W"], bx1=p["x1"]["b"][None, :],
        gx=p["x_ln"]["g"][None, :], bex=p["x_ln"]["b"][None, :],
        wx2=p["x2"]["W"], bx2=p["x2"]["b"][None, :],
    )
    grid = (E // BE,)
    eb = lambda f: pl.BlockSpec((BE, f), lambda i: (i, 0))
    full = lambda a: pl.BlockSpec(a.shape, lambda i: (0,) * a.ndim)
    return pl.pallas_call(
        _edge_kernel,
        grid=grid,
        in_specs=[eb(HID), eb(HID), eb(XP), eb(XP), eb(A_NF)]
                 + [full(v) for v in ops.values()],
        out_specs=[eb(XP), eb(HID)],
        out_shape=[jax.ShapeDtypeStruct((E, XP), jnp.float32),
                   jax.ShapeDtypeStruct((E, HID), jnp.float32)],
    )(hi, hj, xi, xj, ea, *ops.values())


def _node_kernel(h_ref, x_ref, a016_ref, a116_ref, a032_ref, a132_ref, c_ref,
                 wh1h_ref, wh1m_ref, bh1_ref, gh_ref, beh_ref,
                 wh2_ref, bh2_ref,
                 xo_ref, ho_ref):
    h = h_ref[...]
    mh_a = a032_ref[...] + a132_ref[...]
    mx_a = a016_ref[...] + a116_ref[...]
    z = (jnp.dot(h, wh1h_ref[...], preferred_element_type=jnp.float32)
         + jnp.dot(mh_a, wh1m_ref[...], preferred_element_type=jnp.float32)
         + bh1_ref[...])
    z = _silu(_ln(z, gh_ref[...], beh_ref[...]))
    ho_ref[...] = (jnp.dot(z, wh2_ref[...], preferred_element_type=jnp.float32)
                   + bh2_ref[...] + h)
    xo_ref[...] = x_ref[...] + mx_a / c_ref[...]


def _node_mlp(h, x, a16, a32, c, p):
    wh1 = p["h1"]["W"]
    ops = dict(
        wh1h=wh1[:HID], wh1m=wh1[HID:], bh1=p["h1"]["b"][None, :],
        gh=p["h_ln"]["g"][None, :], beh=p["h_ln"]["b"][None, :],
        wh2=p["h2"]["W"], bh2=p["h2"]["b"][None, :],
    )
    grid = (N // BN,)
    nb = lambda f: pl.BlockSpec((BN, f), lambda i: (i, 0))
    nb3 = lambda f: pl.BlockSpec((1, BN, f), lambda i: (0, i, 0))
    full = lambda a: pl.BlockSpec(a.shape, lambda i: (0,) * a.ndim)
    a016 = a16[0]
    a116 = a16[1]
    a032 = a32[0]
    a132 = a32[1]
    xo, ho = pl.pallas_call(
        _node_kernel,
        grid=grid,
        in_specs=[nb(HID), nb(XP), nb(XP), nb(XP), nb(HID), nb(HID), nb(1)]
                 + [full(v) for v in ops.values()],
        out_specs=[nb(XP), nb(HID)],
        out_shape=[jax.ShapeDtypeStruct((N, XP), jnp.float32),
                   jax.ShapeDtypeStruct((N, HID), jnp.float32)],
    )(h, x, a016, a116, a032, a132, c, *ops.values())
    return xo, ho


def kernel(x, h, edges, edge_attr, params):
    src = edges[0]
    dst = edges[1]
    src3d = src.reshape(E // SCC, SJ, SB)
    dst3d = dst.reshape(E // SCC, SJ, SB)
    ones = jnp.ones((SB, XP), jnp.float32)
    zro16 = jnp.zeros((1000, XP), jnp.float32)
    zro32 = jnp.zeros((1000, HID), jnp.float32)
    cp = _degree_sc(src3d, ones, zro16)
    c = (cp[0, :, :1] + cp[1, :, :1])
    xp = jnp.pad(x, ((0, 0), (0, XP - 3)))
    h = h @ params["emb"]["W"] + params["emb"]["b"]
    for p in params["layers"]:
        hi, hj, xi, xj = _gather_sc(h, xp, dst, src)
        mx, mh = _edge_mlp(hi, hj, xi, xj, edge_attr, p)
        a16, a32 = _scatter_sc(mx, mh, dst3d, zro16, zro32)
        xp, h = _node_mlp(h, xp, a16, a32, c, p)
    h = h @ params["emb_out"]["W"] + params["emb_out"]["b"]
    return (xp[:, :3], h)


# speedup vs baseline: 3.0342x; 3.0342x over previous
import functools

import jax
import jax.numpy as jnp
from jax import lax
from jax.experimental import pallas as pl
from jax.experimental.pallas import tpu as pltpu
from jax.experimental.pallas import tpu_sc as plsc

N = 10000
E = 640000
IN_NF = 128
HID = 32
A_NF = 16
PK = 4
E4 = E // PK
BE4 = 1600
BN = 2000

NC = 2
NS = 16
NW = NC * NS
EPW = E // NW
GC = 800
GK = 80
SB = 80
SJ = 10
SCC = SB * SJ


def _gather_sc(ht, xt, dst, src):
    mesh = plsc.VectorSubcoreMesh(core_axis_name="c", subcore_axis_name="s")

    @functools.partial(
        pl.kernel, mesh=mesh,
        out_type=[jax.ShapeDtypeStruct((E, HID), jnp.float32),
                  jax.ShapeDtypeStruct((E, HID), jnp.float32),
                  jax.ShapeDtypeStruct((E, HID), jnp.float32),
                  jax.ShapeDtypeStruct((E, HID), jnp.float32)],
        scratch_types=[pltpu.VMEM((GC,), jnp.int32),
                       pltpu.VMEM((GC,), jnp.int32),
                       pltpu.VMEM((GC, HID), jnp.float32),
                       pltpu.VMEM((GC, HID), jnp.float32),
                       pltpu.VMEM((GC, HID), jnp.float32),
                       pltpu.VMEM((GC, HID), jnp.float32),
                       pltpu.SemaphoreType.DMA],
        compiler_params=pltpu.CompilerParams(use_tc_tiling_on_sc=False),
    )
    def k(ht_hbm, xt_hbm, dst_hbm, src_hbm,
          hi_hbm, hj_hbm, xi_hbm, xj_hbm,
          idxd_v, idxs_v, hd_v, hs_v, xd_v, xs_v, sem):
        wid = lax.axis_index("s") * NC + lax.axis_index("c")
        base0 = wid * EPW

        def body(t, carry):
            base = base0 + t * GC
            pltpu.sync_copy(dst_hbm.at[pl.ds(base, GC)], idxd_v)
            pltpu.sync_copy(src_hbm.at[pl.ds(base, GC)], idxs_v)
            cps = []
            for j in range(GC // GK):
                s = pl.ds(j * GK, GK)
                cps.append(pltpu.async_copy(ht_hbm.at[idxd_v.at[s]], hd_v.at[s], sem))
                cps.append(pltpu.async_copy(ht_hbm.at[idxs_v.at[s]], hs_v.at[s], sem))
                cps.append(pltpu.async_copy(xt_hbm.at[idxd_v.at[s]], xd_v.at[s], sem))
                cps.append(pltpu.async_copy(xt_hbm.at[idxs_v.at[s]], xs_v.at[s], sem))
            for cp in cps:
                cp.wait()
            pltpu.sync_copy(hd_v, hi_hbm.at[pl.ds(base, GC)])
            pltpu.sync_copy(hs_v, hj_hbm.at[pl.ds(base, GC)])
            pltpu.sync_copy(xd_v, xi_hbm.at[pl.ds(base, GC)])
            pltpu.sync_copy(xs_v, xj_hbm.at[pl.ds(base, GC)])
            return carry

        lax.fori_loop(0, EPW // GC, body, 0)

    return k(ht, xt, dst, src)


def _scatter_sc(mx, mh, dst3d, zro32):
    mesh = plsc.VectorSubcoreMesh(core_axis_name="c", subcore_axis_name="s")

    @functools.partial(
        pl.kernel, mesh=mesh,
        out_type=[jax.ShapeDtypeStruct((NC, N, HID), jnp.float32),
                  jax.ShapeDtypeStruct((NC, N, HID), jnp.float32)],
        scratch_types=[pltpu.VMEM((SJ, SB), jnp.int32),
                       pltpu.VMEM((SCC, HID), jnp.float32),
                       pltpu.VMEM((SCC, HID), jnp.float32),
                       pltpu.VMEM_SHARED((N, HID), jnp.float32),
                       pltpu.VMEM_SHARED((N, HID), jnp.float32)],
        compiler_params=pltpu.CompilerParams(use_tc_tiling_on_sc=False),
    )
    def k(mx_hbm, mh_hbm, dst_hbm, z32_hbm, ox_hbm, oh_hbm,
          idx_v, mx_v, mh_v, shx, shh):
        cid = lax.axis_index("c")
        sid = lax.axis_index("s")
        wid = sid * NC + cid

        @pl.when(sid < 10)
        def _():
            pltpu.sync_copy(z32_hbm, shx.at[pl.ds(sid * 1000, 1000)])
            pltpu.sync_copy(z32_hbm, shh.at[pl.ds(sid * 1000, 1000)])

        plsc.subcore_barrier()

        def body(t, carry):
            blk = wid * (EPW // SCC) + t
            base = wid * EPW + t * SCC
            pltpu.sync_copy(dst_hbm.at[blk], idx_v)
            pltpu.sync_copy(mx_hbm.at[pl.ds(base, SCC)], mx_v)
            pltpu.sync_copy(mh_hbm.at[pl.ds(base, SCC)], mh_v)
            for j in range(SJ):
                s = pl.ds(j * SB, SB)
                pltpu.sync_copy(mx_v.at[s], shx.at[idx_v.at[j]], add=True)
                pltpu.sync_copy(mh_v.at[s], shh.at[idx_v.at[j]], add=True)
            return carry

        lax.fori_loop(0, EPW // SCC, body, 0)
        plsc.subcore_barrier()

        @pl.when(sid < 10)
        def _():
            pltpu.sync_copy(shx.at[pl.ds(sid * 1000, 1000)],
                            ox_hbm.at[cid, pl.ds(sid * 1000, 1000)])
            pltpu.sync_copy(shh.at[pl.ds(sid * 1000, 1000)],
                            oh_hbm.at[cid, pl.ds(sid * 1000, 1000)])

    return k(mx, mh, dst3d, zro32)


def _degree_sc(src3d, ones, zro32):
    mesh = plsc.VectorSubcoreMesh(core_axis_name="c", subcore_axis_name="s")

    @functools.partial(
        pl.kernel, mesh=mesh,
        out_type=jax.ShapeDtypeStruct((NC, N, HID), jnp.float32),
        scratch_types=[pltpu.VMEM((SJ, SB), jnp.int32),
                       pltpu.VMEM((SB, HID), jnp.float32),
                       pltpu.VMEM_SHARED((N, HID), jnp.float32)],
        compiler_params=pltpu.CompilerParams(use_tc_tiling_on_sc=False),
    )
    def k(src_hbm, ones_hbm, zro_hbm, out_hbm, idx_v, ones_v, shared):
        cid = lax.axis_index("c")
        sid = lax.axis_index("s")
        wid = sid * NC + cid
        pltpu.sync_copy(ones_hbm, ones_v)

        @pl.when(sid < 10)
        def _():
            pltpu.sync_copy(zro_hbm, shared.at[pl.ds(sid * 1000, 1000)])

        plsc.subcore_barrier()

        def body(t, carry):
            blk = wid * (EPW // SCC) + t
            pltpu.sync_copy(src_hbm.at[blk], idx_v)
            for j in range(SJ):
                pltpu.sync_copy(ones_v, shared.at[idx_v.at[j]], add=True)
            return carry

        lax.fori_loop(0, EPW // SCC, body, 0)
        plsc.subcore_barrier()

        @pl.when(sid < 10)
        def _():
            pltpu.sync_copy(shared.at[pl.ds(sid * 1000, 1000)],
                            out_hbm.at[cid, pl.ds(sid * 1000, 1000)])

    return k(src3d, ones, zro32)


def _sigmoid(z):
    return 1.0 / (1.0 + jnp.exp(-z))


def _silu(z):
    return z * _sigmoid(z)


def _ln4(z, g, b, pmu_ref, eps=1e-5):
    mu = jnp.dot(z, pmu_ref[...], preferred_element_type=jnp.float32)
    msq = jnp.dot(z * z, pmu_ref[...], preferred_element_type=jnp.float32)
    var = msq - mu * mu
    return (z - mu) * jax.lax.rsqrt(var + eps) * g + b


def _edge_kernel(hi_ref, hj_ref, xi_ref, xj_ref, ea_ref,
                 w1hi_ref, w1hj_ref, w1ea_ref, w1d2_ref, b1_ref, g1_ref, be1_ref,
                 w2_ref, b2_ref, g2_ref, be2_ref,
                 wx1_ref, bx1_ref, gx_ref, bex_ref, bpx_ref, bx2_ref,
                 ps_ref, pmu_ref,
                 mx_ref, mh_ref):
    diff = xi_ref[...] - xj_ref[...]
    d2 = jnp.dot(diff * diff, ps_ref[...], preferred_element_type=jnp.float32)
    z = (jnp.dot(hi_ref[...], w1hi_ref[...], preferred_element_type=jnp.float32)
         + jnp.dot(hj_ref[...], w1hj_ref[...], preferred_element_type=jnp.float32)
         + jnp.dot(ea_ref[...], w1ea_ref[...], preferred_element_type=jnp.float32)
         + d2 * w1d2_ref[...]
         + b1_ref[...])
    z = _silu(_ln4(z, g1_ref[...], be1_ref[...], pmu_ref))
    z = jnp.dot(z, w2_ref[...], preferred_element_type=jnp.float32) + b2_ref[...]
    mh = _silu(_ln4(z, g2_ref[...], be2_ref[...], pmu_ref))
    t = jnp.dot(mh, wx1_ref[...], preferred_element_type=jnp.float32) + bx1_ref[...]
    t = _silu(_ln4(t, gx_ref[...], bex_ref[...], pmu_ref))
    px = jnp.dot(t, bpx_ref[...], preferred_element_type=jnp.float32) + bx2_ref[...]
    mx_ref[...] = diff * px
    mh_ref[...] = mh


def _tile4(v):
    return jnp.tile(v[None, :], (1, PK)).reshape(1, PK * HID)


def _edge_mlp(hi4, hj4, xi4, xj4, ea4, p):
    eye = jnp.eye(PK, dtype=jnp.float32)
    w1 = p["e1"]["W"]
    w1ea = jnp.zeros((HID, HID), jnp.float32).at[:A_NF].set(w1[2 * HID + 1:])
    wx2 = p["x2"]["W"]
    ops = dict(
        w1hi=jnp.kron(eye, w1[:HID]), w1hj=jnp.kron(eye, w1[HID:2 * HID]),
        w1ea=jnp.kron(eye, w1ea),
        w1d2=_tile4(w1[2 * HID]), b1=_tile4(p["e1"]["b"]),
        g1=_tile4(p["e_ln1"]["g"]), be1=_tile4(p["e_ln1"]["b"]),
        w2=jnp.kron(eye, p["e2"]["W"]), b2=_tile4(p["e2"]["b"]),
        g2=_tile4(p["e_ln2"]["g"]), be2=_tile4(p["e_ln2"]["b"]),
        wx1=jnp.kron(eye, p["x1"]["W"]), bx1=_tile4(p["x1"]["b"]),
        gx=_tile4(p["x_ln"]["g"]), bex=_tile4(p["x_ln"]["b"]),
        bpx=jnp.kron(eye, wx2 @ jnp.ones((1, HID), jnp.float32)),
        bx2=jnp.full((1, PK * HID), p["x2"]["b"][0], jnp.float32),
        ps=jnp.kron(eye, jnp.ones((HID, HID), jnp.float32)),
        pmu=jnp.kron(eye, jnp.full((HID, HID), 1.0 / HID, jnp.float32)),
    )
    grid = (E4 // BE4,)
    eb = pl.BlockSpec((BE4, PK * HID), lambda i: (i, 0))
    full = lambda a: pl.BlockSpec(a.shape, lambda i: (0,) * a.ndim)
    return pl.pallas_call(
        _edge_kernel,
        grid=grid,
        in_specs=[eb, eb, eb, eb, eb] + [full(v) for v in ops.values()],
        out_specs=[eb, eb],
        out_shape=[jax.ShapeDtypeStruct((E4, PK * HID), jnp.float32),
                   jax.ShapeDtypeStruct((E4, PK * HID), jnp.float32)],
    )(hi4, hj4, xi4, xj4, ea4, *ops.values())


def _node_kernel(h_ref, x_ref, a0x_ref, a1x_ref, a0h_ref, a1h_ref, c_ref,
                 wh1h_ref, wh1m_ref, bh1_ref, gh_ref, beh_ref,
                 wh2_ref, bh2_ref,
                 xo_ref, ho_ref):
    h = h_ref[...]
    mh_a = a0h_ref[...] + a1h_ref[...]
    mx_a = a0x_ref[...] + a1x_ref[...]
    z = (jnp.dot(h, wh1h_ref[...], preferred_element_type=jnp.float32)
         + jnp.dot(mh_a, wh1m_ref[...], preferred_element_type=jnp.float32)
         + bh1_ref[...])
    z = _silu(_ln(z, gh_ref[...], beh_ref[...]))
    ho_ref[...] = (jnp.dot(z, wh2_ref[...], preferred_element_type=jnp.float32)
                   + bh2_ref[...] + h)
    xo_ref[...] = x_ref[...] + mx_a / c_ref[...]


def _ln(z, g, b, eps=1e-5):
    o = jnp.full((HID, 1), 1.0 / HID, jnp.float32)
    mu = jnp.dot(z, o, preferred_element_type=jnp.float32)
    msq = jnp.dot(z * z, o, preferred_element_type=jnp.float32)
    var = msq - mu * mu
    return (z - mu) * jax.lax.rsqrt(var + eps) * g + b


def _node_mlp(h, x, ax, ah, c, p):
    wh1 = p["h1"]["W"]
    ops = dict(
        wh1h=wh1[:HID], wh1m=wh1[HID:], bh1=p["h1"]["b"][None, :],
        gh=p["h_ln"]["g"][None, :], beh=p["h_ln"]["b"][None, :],
        wh2=p["h2"]["W"], bh2=p["h2"]["b"][None, :],
    )
    grid = (N // BN,)
    nb = pl.BlockSpec((BN, HID), lambda i: (i, 0))
    nc = pl.BlockSpec((BN, 1), lambda i: (i, 0))
    full = lambda a: pl.BlockSpec(a.shape, lambda i: (0,) * a.ndim)
    xo, ho = pl.pallas_call(
        _node_kernel,
        grid=grid,
        in_specs=[nb, nb, nb, nb, nb, nb, nc]
                 + [full(v) for v in ops.values()],
        out_specs=[nb, nb],
        out_shape=[jax.ShapeDtypeStruct((N, HID), jnp.float32),
                   jax.ShapeDtypeStruct((N, HID), jnp.float32)],
    )(h, x, ax[0], ax[1], ah[0], ah[1], c, *ops.values())
    return xo, ho


def kernel(x, h, edges, edge_attr, params):
    src = edges[0]
    dst = edges[1]
    src3d = src.reshape(E // SCC, SJ, SB)
    dst3d = dst.reshape(E // SCC, SJ, SB)
    ones = jnp.ones((SB, HID), jnp.float32)
    zro32 = jnp.zeros((1000, HID), jnp.float32)
    cp = _degree_sc(src3d, ones, zro32)
    c = (cp[0, :, :1] + cp[1, :, :1])
    xp = jnp.pad(x, ((0, 0), (0, HID - 3)))
    ea4 = jnp.pad(edge_attr, ((0, 0), (0, HID - A_NF))).reshape(E4, PK * HID)
    h = h @ params["emb"]["W"] + params["emb"]["b"]
    for p in params["layers"]:
        hi, hj, xi, xj = _gather_sc(h, xp, dst, src)
        mx4, mh4 = _edge_mlp(hi.reshape(E4, PK * HID), hj.reshape(E4, PK * HID),
                             xi.reshape(E4, PK * HID), xj.reshape(E4, PK * HID),
                             ea4, p)
        ax, ah = _scatter_sc(mx4.reshape(E, HID), mh4.reshape(E, HID),
                             dst3d, zro32)
        xp, h = _node_mlp(h, xp, ax, ah, c, p)
    h = h @ params["emb_out"]["W"] + params["emb_out"]["b"]
    return (xp[:, :3], h)


# speedup vs baseline: 3.4226x; 1.1280x over previous
import functools

import jax
import jax.numpy as jnp
from jax import lax
from jax.experimental import pallas as pl
from jax.experimental.pallas import tpu as pltpu
from jax.experimental.pallas import tpu_sc as plsc

N = 10000
E = 640000
IN_NF = 128
HID = 32
A_NF = 16
PK = 4
E4 = E // PK
BE4 = 1600
BN = 2000

NC = 2
NS = 16
NW = NC * NS
EPW = E // NW
GC = 400
GK = 80
SB = 80
SJ = 5
SCC = SB * SJ


def _gather_sc(ht, xt, dst, src):
    mesh = plsc.VectorSubcoreMesh(core_axis_name="c", subcore_axis_name="s")

    @functools.partial(
        pl.kernel, mesh=mesh,
        out_type=[jax.ShapeDtypeStruct((E, HID), jnp.float32),
                  jax.ShapeDtypeStruct((E, HID), jnp.float32),
                  jax.ShapeDtypeStruct((E, HID), jnp.float32),
                  jax.ShapeDtypeStruct((E, HID), jnp.float32)],
        scratch_types=[pltpu.VMEM((2, GC), jnp.int32),
                       pltpu.VMEM((2, GC), jnp.int32),
                       pltpu.VMEM((2, GC, HID), jnp.float32),
                       pltpu.VMEM((2, GC, HID), jnp.float32),
                       pltpu.VMEM((2, GC, HID), jnp.float32),
                       pltpu.VMEM((2, GC, HID), jnp.float32),
                       pltpu.SemaphoreType.DMA,
                       pltpu.SemaphoreType.DMA((2,)),
                       pltpu.SemaphoreType.DMA((2,))],
        compiler_params=pltpu.CompilerParams(use_tc_tiling_on_sc=False),
    )
    def k(ht_hbm, xt_hbm, dst_hbm, src_hbm,
          hi_hbm, hj_hbm, xi_hbm, xj_hbm,
          idxd_v, idxs_v, hd_v, hs_v, xd_v, xs_v, gsem, isem, ssem):
        wid = lax.axis_index("s") * NC + lax.axis_index("c")
        base0 = wid * EPW
        nchunk = EPW // GC

        def issue_idx(cidx, b):
            base = base0 + cidx * GC
            pltpu.async_copy(dst_hbm.at[pl.ds(base, GC)], idxd_v.at[b], isem.at[b])
            pltpu.async_copy(src_hbm.at[pl.ds(base, GC)], idxs_v.at[b], isem.at[b])

        issue_idx(0, 0)
        issue_idx(1, 1)

        def chunk(cidx, b):
            base = base0 + cidx * GC
            outs = ((hd_v, hi_hbm), (hs_v, hj_hbm), (xd_v, xi_hbm), (xs_v, xj_hbm))
            pltpu.make_async_copy(dst_hbm.at[pl.ds(base, GC)], idxd_v.at[b],
                                  isem.at[b]).wait()
            pltpu.make_async_copy(src_hbm.at[pl.ds(base, GC)], idxs_v.at[b],
                                  isem.at[b]).wait()

            @pl.when(cidx >= 2)
            def _():
                for v, o in outs:
                    pltpu.make_async_copy(v.at[b], o.at[pl.ds(base, GC)],
                                          ssem.at[b]).wait()

            cps = []
            for j in range(GC // GK):
                s = pl.ds(j * GK, GK)
                cps.append(pltpu.async_copy(ht_hbm.at[idxd_v.at[b, s]],
                                            hd_v.at[b, s], gsem))
                cps.append(pltpu.async_copy(ht_hbm.at[idxs_v.at[b, s]],
                                            hs_v.at[b, s], gsem))
                cps.append(pltpu.async_copy(xt_hbm.at[idxd_v.at[b, s]],
                                            xd_v.at[b, s], gsem))
                cps.append(pltpu.async_copy(xt_hbm.at[idxs_v.at[b, s]],
                                            xs_v.at[b, s], gsem))
            for cp in cps:
                cp.wait()

            @pl.when(cidx + 2 < nchunk)
            def _():
                issue_idx(cidx + 2, b)

            for v, o in outs:
                pltpu.async_copy(v.at[b], o.at[pl.ds(base, GC)], ssem.at[b])

        def body(t, carry):
            chunk(2 * t, 0)
            chunk(2 * t + 1, 1)
            return carry

        lax.fori_loop(0, nchunk // 2, body, 0)
        for b in (0, 1):
            for v, o in ((hd_v, hi_hbm), (hs_v, hj_hbm), (xd_v, xi_hbm),
                         (xs_v, xj_hbm)):
                pltpu.make_async_copy(v.at[b], o.at[pl.ds(base0, GC)],
                                      ssem.at[b]).wait()

    return k(ht, xt, dst, src)


def _scatter_sc(mx, mh, dst3d, zro32):
    mesh = plsc.VectorSubcoreMesh(core_axis_name="c", subcore_axis_name="s")

    @functools.partial(
        pl.kernel, mesh=mesh,
        out_type=[jax.ShapeDtypeStruct((NC, N, HID), jnp.float32),
                  jax.ShapeDtypeStruct((NC, N, HID), jnp.float32)],
        scratch_types=[pltpu.VMEM((2, SJ, SB), jnp.int32),
                       pltpu.VMEM((2, SCC, HID), jnp.float32),
                       pltpu.VMEM((2, SCC, HID), jnp.float32),
                       pltpu.VMEM_SHARED((N, HID), jnp.float32),
                       pltpu.VMEM_SHARED((N, HID), jnp.float32),
                       pltpu.SemaphoreType.DMA,
                       pltpu.SemaphoreType.DMA((2,))],
        compiler_params=pltpu.CompilerParams(use_tc_tiling_on_sc=False),
    )
    def k(mx_hbm, mh_hbm, dst_hbm, z32_hbm, ox_hbm, oh_hbm,
          idx_v, mx_v, mh_v, shx, shh, asem, lsem):
        cid = lax.axis_index("c")
        sid = lax.axis_index("s")
        wid = sid * NC + cid
        nchunk = EPW // SCC

        @pl.when(sid < 10)
        def _():
            pltpu.sync_copy(z32_hbm, shx.at[pl.ds(sid * 1000, 1000)])
            pltpu.sync_copy(z32_hbm, shh.at[pl.ds(sid * 1000, 1000)])

        plsc.subcore_barrier()

        def issue_loads(cidx, b):
            blk = wid * nchunk + cidx
            base = wid * EPW + cidx * SCC
            pltpu.async_copy(dst_hbm.at[blk], idx_v.at[b], lsem.at[b])
            pltpu.async_copy(mx_hbm.at[pl.ds(base, SCC)], mx_v.at[b], lsem.at[b])
            pltpu.async_copy(mh_hbm.at[pl.ds(base, SCC)], mh_v.at[b], lsem.at[b])

        issue_loads(0, 0)
        issue_loads(1, 1)

        def chunk(cidx, b):
            blk = wid * nchunk + cidx
            base = wid * EPW + cidx * SCC
            pltpu.make_async_copy(dst_hbm.at[blk], idx_v.at[b], lsem.at[b]).wait()
            pltpu.make_async_copy(mx_hbm.at[pl.ds(base, SCC)], mx_v.at[b],
                                  lsem.at[b]).wait()
            pltpu.make_async_copy(mh_hbm.at[pl.ds(base, SCC)], mh_v.at[b],
                                  lsem.at[b]).wait()
            cps = []
            for j in range(SJ):
                s = pl.ds(j * SB, SB)
                cps.append(pltpu.async_copy(mx_v.at[b, s], shx.at[idx_v.at[b, j]],
                                            asem, add=True))
                cps.append(pltpu.async_copy(mh_v.at[b, s], shh.at[idx_v.at[b, j]],
                                            asem, add=True))
            for cp in cps:
                cp.wait()

            @pl.when(cidx + 2 < nchunk)
            def _():
                issue_loads(cidx + 2, b)

        def body(t, carry):
            chunk(2 * t, 0)
            chunk(2 * t + 1, 1)
            return carry

        lax.fori_loop(0, nchunk // 2, body, 0)
        plsc.subcore_barrier()

        @pl.when(sid < 10)
        def _():
            pltpu.sync_copy(shx.at[pl.ds(sid * 1000, 1000)],
                            ox_hbm.at[cid, pl.ds(sid * 1000, 1000)])
            pltpu.sync_copy(shh.at[pl.ds(sid * 1000, 1000)],
                            oh_hbm.at[cid, pl.ds(sid * 1000, 1000)])

    return k(mx, mh, dst3d, zro32)


def _degree_sc(src3d, ones, zro32):
    mesh = plsc.VectorSubcoreMesh(core_axis_name="c", subcore_axis_name="s")

    @functools.partial(
        pl.kernel, mesh=mesh,
        out_type=jax.ShapeDtypeStruct((NC, N, HID), jnp.float32),
        scratch_types=[pltpu.VMEM((SJ, SB), jnp.int32),
                       pltpu.VMEM((SB, HID), jnp.float32),
                       pltpu.VMEM_SHARED((N, HID), jnp.float32)],
        compiler_params=pltpu.CompilerParams(use_tc_tiling_on_sc=False),
    )
    def k(src_hbm, ones_hbm, zro_hbm, out_hbm, idx_v, ones_v, shared):
        cid = lax.axis_index("c")
        sid = lax.axis_index("s")
        wid = sid * NC + cid
        pltpu.sync_copy(ones_hbm, ones_v)

        @pl.when(sid < 10)
        def _():
            pltpu.sync_copy(zro_hbm, shared.at[pl.ds(sid * 1000, 1000)])

        plsc.subcore_barrier()

        def body(t, carry):
            blk = wid * (EPW // SCC) + t
            pltpu.sync_copy(src_hbm.at[blk], idx_v)
            for j in range(SJ):
                pltpu.sync_copy(ones_v, shared.at[idx_v.at[j]], add=True)
            return carry

        lax.fori_loop(0, EPW // SCC, body, 0)
        plsc.subcore_barrier()

        @pl.when(sid < 10)
        def _():
            pltpu.sync_copy(shared.at[pl.ds(sid * 1000, 1000)],
                            out_hbm.at[cid, pl.ds(sid * 1000, 1000)])

    return k(src3d, ones, zro32)


def _sigmoid(z):
    return 1.0 / (1.0 + jnp.exp(-z))


def _silu(z):
    return z * _sigmoid(z)


def _ln4(z, g, b, pmu_ref, eps=1e-5):
    mu = jnp.dot(z, pmu_ref[...], preferred_element_type=jnp.float32)
    msq = jnp.dot(z * z, pmu_ref[...], preferred_element_type=jnp.float32)
    var = msq - mu * mu
    return (z - mu) * jax.lax.rsqrt(var + eps) * g + b


def _edge_kernel(hi_ref, hj_ref, xi_ref, xj_ref, ea_ref,
                 w1hi_ref, w1hj_ref, w1ea_ref, w1d2_ref, b1_ref, g1_ref, be1_ref,
                 w2_ref, b2_ref, g2_ref, be2_ref,
                 wx1_ref, bx1_ref, gx_ref, bex_ref, bpx_ref, bx2_ref,
                 ps_ref, pmu_ref,
                 mx_ref, mh_ref):
    diff = xi_ref[...] - xj_ref[...]
    d2 = jnp.dot(diff * diff, ps_ref[...], preferred_element_type=jnp.float32)
    z = (jnp.dot(hi_ref[...], w1hi_ref[...], preferred_element_type=jnp.float32)
         + jnp.dot(hj_ref[...], w1hj_ref[...], preferred_element_type=jnp.float32)
         + jnp.dot(ea_ref[...], w1ea_ref[...], preferred_element_type=jnp.float32)
         + d2 * w1d2_ref[...]
         + b1_ref[...])
    z = _silu(_ln4(z, g1_ref[...], be1_ref[...], pmu_ref))
    z = jnp.dot(z, w2_ref[...], preferred_element_type=jnp.float32) + b2_ref[...]
    mh = _silu(_ln4(z, g2_ref[...], be2_ref[...], pmu_ref))
    t = jnp.dot(mh, wx1_ref[...], preferred_element_type=jnp.float32) + bx1_ref[...]
    t = _silu(_ln4(t, gx_ref[...], bex_ref[...], pmu_ref))
    px = jnp.dot(t, bpx_ref[...], preferred_element_type=jnp.float32) + bx2_ref[...]
    mx_ref[...] = diff * px
    mh_ref[...] = mh


def _tile4(v):
    return jnp.tile(v[None, :], (1, PK)).reshape(1, PK * HID)


def _edge_mlp(hi4, hj4, xi4, xj4, ea4, p):
    eye = jnp.eye(PK, dtype=jnp.float32)
    w1 = p["e1"]["W"]
    w1ea = jnp.zeros((HID, HID), jnp.float32).at[:A_NF].set(w1[2 * HID + 1:])
    wx2 = p["x2"]["W"]
    ops = dict(
        w1hi=jnp.kron(eye, w1[:HID]), w1hj=jnp.kron(eye, w1[HID:2 * HID]),
        w1ea=jnp.kron(eye, w1ea),
        w1d2=_tile4(w1[2 * HID]), b1=_tile4(p["e1"]["b"]),
        g1=_tile4(p["e_ln1"]["g"]), be1=_tile4(p["e_ln1"]["b"]),
        w2=jnp.kron(eye, p["e2"]["W"]), b2=_tile4(p["e2"]["b"]),
        g2=_tile4(p["e_ln2"]["g"]), be2=_tile4(p["e_ln2"]["b"]),
        wx1=jnp.kron(eye, p["x1"]["W"]), bx1=_tile4(p["x1"]["b"]),
        gx=_tile4(p["x_ln"]["g"]), bex=_tile4(p["x_ln"]["b"]),
        bpx=jnp.kron(eye, wx2 @ jnp.ones((1, HID), jnp.float32)),
        bx2=jnp.full((1, PK * HID), p["x2"]["b"][0], jnp.float32),
        ps=jnp.kron(eye, jnp.ones((HID, HID), jnp.float32)),
        pmu=jnp.kron(eye, jnp.full((HID, HID), 1.0 / HID, jnp.float32)),
    )
    grid = (E4 // BE4,)
    eb = pl.BlockSpec((BE4, PK * HID), lambda i: (i, 0))
    full = lambda a: pl.BlockSpec(a.shape, lambda i: (0,) * a.ndim)
    return pl.pallas_call(
        _edge_kernel,
        grid=grid,
        in_specs=[eb, eb, eb, eb, eb] + [full(v) for v in ops.values()],
        out_specs=[eb, eb],
        out_shape=[jax.ShapeDtypeStruct((E4, PK * HID), jnp.float32),
                   jax.ShapeDtypeStruct((E4, PK * HID), jnp.float32)],
    )(hi4, hj4, xi4, xj4, ea4, *ops.values())


def _node_kernel(h_ref, x_ref, a0x_ref, a1x_ref, a0h_ref, a1h_ref, c_ref,
                 wh1h_ref, wh1m_ref, bh1_ref, gh_ref, beh_ref,
                 wh2_ref, bh2_ref,
                 xo_ref, ho_ref):
    h = h_ref[...]
    mh_a = a0h_ref[...] + a1h_ref[...]
    mx_a = a0x_ref[...] + a1x_ref[...]
    z = (jnp.dot(h, wh1h_ref[...], preferred_element_type=jnp.float32)
         + jnp.dot(mh_a, wh1m_ref[...], preferred_element_type=jnp.float32)
         + bh1_ref[...])
    z = _silu(_ln(z, gh_ref[...], beh_ref[...]))
    ho_ref[...] = (jnp.dot(z, wh2_ref[...], preferred_element_type=jnp.float32)
                   + bh2_ref[...] + h)
    xo_ref[...] = x_ref[...] + mx_a / c_ref[...]


def _ln(z, g, b, eps=1e-5):
    o = jnp.full((HID, 1), 1.0 / HID, jnp.float32)
    mu = jnp.dot(z, o, preferred_element_type=jnp.float32)
    msq = jnp.dot(z * z, o, preferred_element_type=jnp.float32)
    var = msq - mu * mu
    return (z - mu) * jax.lax.rsqrt(var + eps) * g + b


def _node_mlp(h, x, ax, ah, c, p):
    wh1 = p["h1"]["W"]
    ops = dict(
        wh1h=wh1[:HID], wh1m=wh1[HID:], bh1=p["h1"]["b"][None, :],
        gh=p["h_ln"]["g"][None, :], beh=p["h_ln"]["b"][None, :],
        wh2=p["h2"]["W"], bh2=p["h2"]["b"][None, :],
    )
    grid = (N // BN,)
    nb = pl.BlockSpec((BN, HID), lambda i: (i, 0))
    nc = pl.BlockSpec((BN, 1), lambda i: (i, 0))
    full = lambda a: pl.BlockSpec(a.shape, lambda i: (0,) * a.ndim)
    xo, ho = pl.pallas_call(
        _node_kernel,
        grid=grid,
        in_specs=[nb, nb, nb, nb, nb, nb, nc]
                 + [full(v) for v in ops.values()],
        out_specs=[nb, nb],
        out_shape=[jax.ShapeDtypeStruct((N, HID), jnp.float32),
                   jax.ShapeDtypeStruct((N, HID), jnp.float32)],
    )(h, x, ax[0], ax[1], ah[0], ah[1], c, *ops.values())
    return xo, ho


def kernel(x, h, edges, edge_attr, params):
    src = edges[0]
    dst = edges[1]
    src3d = src.reshape(E // SCC, SJ, SB)
    dst3d = dst.reshape(E // SCC, SJ, SB)
    ones = jnp.ones((SB, HID), jnp.float32)
    zro32 = jnp.zeros((1000, HID), jnp.float32)
    cp = _degree_sc(src3d, ones, zro32)
    c = (cp[0, :, :1] + cp[1, :, :1])
    xp = jnp.pad(x, ((0, 0), (0, HID - 3)))
    ea4 = jnp.pad(edge_attr, ((0, 0), (0, HID - A_NF))).reshape(E4, PK * HID)
    h = h @ params["emb"]["W"] + params["emb"]["b"]
    for p in params["layers"]:
        hi, hj, xi, xj = _gather_sc(h, xp, dst, src)
        mx4, mh4 = _edge_mlp(hi.reshape(E4, PK * HID), hj.reshape(E4, PK * HID),
                             xi.reshape(E4, PK * HID), xj.reshape(E4, PK * HID),
                             ea4, p)
        ax, ah = _scatter_sc(mx4.reshape(E, HID), mh4.reshape(E, HID),
                             dst3d, zro32)
        xp, h = _node_mlp(h, xp, ax, ah, c, p)
    h = h @ params["emb_out"]["W"] + params["emb_out"]["b"]
    return (xp[:, :3], h)
